# baseline (speedup 1.0000x reference)
import functools

import jax
import jax.numpy as jnp
from jax import lax
from jax.experimental import pallas as pl
from jax.experimental.pallas import tpu as pltpu
from jax.experimental.pallas import tpu_sc as plsc

_GSZ = 80
_NB = 5


def _spmm_sc(h, src, dst, w, n, d):
    e = src.shape[0]
    info = plsc.get_sparse_core_info()
    nc, ns = info.num_cores, info.num_subcores
    nw = nc * ns
    ngrp = e // (nw * _GSZ)
    rows_per_tile = 1000
    num_copy_tiles = n // rows_per_tile
    zeros = jnp.zeros((n, d), jnp.float32)
    src2 = src.reshape(e // _GSZ, _GSZ)
    dst2 = dst.reshape(e // _GSZ, _GSZ)
    w2 = w.reshape(e // _GSZ, _GSZ)
    mesh = plsc.VectorSubcoreMesh(core_axis_name="c", subcore_axis_name="s")

    @functools.partial(
        pl.kernel,
        mesh=mesh,
        out_type=jax.ShapeDtypeStruct((nc, n, d), jnp.float32),
        scratch_types=[
            pltpu.VMEM((ngrp, _GSZ), jnp.int32),
            pltpu.VMEM((ngrp, _GSZ), jnp.int32),
            pltpu.VMEM((ngrp, _GSZ), jnp.float32),
            [pltpu.VMEM((_GSZ, d), jnp.float32) for _ in range(_NB)],
            [pltpu.SemaphoreType.DMA for _ in range(_NB)],
            pltpu.VMEM_SHARED((n, d), jnp.float32),
        ],
        compiler_params=pltpu.CompilerParams(use_tc_tiling_on_sc=False),
    )
    def spmm(h_hbm, src_hbm, dst_hbm, w_hbm, z_hbm, out_hbm,
             src_st, dst_st, w_st, rows_bufs, sems, acc):
        cid = lax.axis_index("c")
        sid = lax.axis_index("s")
        wid = cid * ns + sid
        t0 = wid * ngrp

        pltpu.sync_copy(src_hbm.at[pl.ds(t0, ngrp)], src_st)
        pltpu.sync_copy(dst_hbm.at[pl.ds(t0, ngrp)], dst_st)
        pltpu.sync_copy(w_hbm.at[pl.ds(t0, ngrp)], w_st)
        for b in range(_NB):
            pltpu.async_copy(h_hbm.at[src_st.at[b]], rows_bufs[b], sems[b])

        r0 = pl.multiple_of(sid * rows_per_tile, 8)

        @pl.when(sid < num_copy_tiles)
        def _zero():
            pltpu.sync_copy(z_hbm.at[pl.ds(r0, rows_per_tile)],
                            acc.at[pl.ds(r0, rows_per_tile)])

        plsc.subcore_barrier()

        def outer(p, carry):
            for b in range(_NB):
                j = p * _NB + b
                rows = rows_bufs[b]
                pltpu.make_async_copy(
                    h_hbm.at[src_st.at[j]], rows, sems[b]).wait()
                for q in range(_GSZ // 16):
                    wv = w_st[j, pl.ds(q * 16, 16)]
                    for ii in range(16):
                        i = q * 16 + ii
                        wsc = wv[ii]
                        for k in range(d // 16):
                            rows[i, pl.ds(k * 16, 16)] = (
                                rows[i, pl.ds(k * 16, 16)] * wsc)
                pltpu.sync_copy(rows, acc.at[dst_st.at[j]], add=True)
                nxt = j + _NB

                @pl.when(nxt < ngrp)
                def _refill():
                    pltpu.async_copy(
                        h_hbm.at[src_st.at[nxt]], rows, sems[b])
            return carry

        lax.fori_loop(0, ngrp // _NB, outer, 0)
        plsc.subcore_barrier()

        @pl.when(sid < num_copy_tiles)
        def _copy_out():
            pltpu.sync_copy(acc.at[pl.ds(r0, rows_per_tile)],
                            out_hbm.at[cid, pl.ds(r0, rows_per_tile)])

    return spmm(h, src2, dst2, w2, zeros)


def _mm1_tc(x, w1):
    n, f = x.shape
    h1 = w1.shape[1]
    bm = 1000

    def body(x_ref, w_ref, o_ref):
        o_ref[...] = jnp.dot(x_ref[...], w_ref[...],
                             preferred_element_type=jnp.float32)

    return pl.pallas_call(
        body,
        grid=(n // bm,),
        in_specs=[
            pl.BlockSpec((bm, f), lambda i: (i, 0)),
            pl.BlockSpec((f, h1), lambda i: (0, 0)),
        ],
        out_specs=pl.BlockSpec((bm, h1), lambda i: (i, 0)),
        out_shape=jax.ShapeDtypeStruct((n, h1), jnp.float32),
    )(x, w1)


def _relu_mm2_tc(p0, p1, w2):
    n, h1 = p0.shape
    h2 = w2.shape[1]

    def body(p0_ref, p1_ref, w_ref, o_ref):
        h = jnp.maximum(p0_ref[...] + p1_ref[...], 0.0)
        o_ref[...] = jnp.dot(h, w_ref[...],
                             preferred_element_type=jnp.float32)

    return pl.pallas_call(
        body,
        out_shape=jax.ShapeDtypeStruct((n, h2), jnp.float32),
    )(p0, p1, w2)


def _decoder_tc(q0, q1):
    n, h2 = q0.shape
    bm = 64
    grid = (n + bm - 1) // bm

    def body(q0b_ref, q1b_ref, q0f_ref, q1f_ref, o_ref, m_ref):
        zb = q0b_ref[...] + q1b_ref[...]
        zf = q0f_ref[...] + q1f_ref[...]
        m_ref[...] = lax.dot_general(
            zb, zf, (((1,), (1,)), ((), ())),
            preferred_element_type=jnp.float32)
        for r in range(bm):
            o_ref[pl.ds(r * n, n)] = m_ref[r, :]

    out = pl.pallas_call(
        body,
        grid=(grid,),
        in_specs=[
            pl.BlockSpec((bm, h2), lambda i: (i, 0)),
            pl.BlockSpec((bm, h2), lambda i: (i, 0)),
            pl.BlockSpec((n, h2), lambda i: (0, 0)),
            pl.BlockSpec((n, h2), lambda i: (0, 0)),
        ],
        out_specs=pl.BlockSpec((bm * n,), lambda i: (i,)),
        out_shape=jax.ShapeDtypeStruct((n * n,), jnp.float32),
        scratch_shapes=[pltpu.VMEM((bm, n), jnp.float32)],
    )(q0, q1, q0, q1)
    return out


def kernel(x, edge_index, edge_weight, W1, W2):
    n = x.shape[0]
    ei = edge_index.astype(jnp.int32)
    src = ei[0]
    dst = ei[1]

    mm1 = _mm1_tc(x, W1)
    p = _spmm_sc(mm1, src, dst, edge_weight, n, W1.shape[1])
    y = _relu_mm2_tc(p[0], p[1], W2)
    q = _spmm_sc(y, src, dst, edge_weight, n, W2.shape[1])
    return _decoder_tc(q[0], q[1])


# speedup vs baseline: 1.2254x; 1.2254x over previous
import functools

import jax
import jax.numpy as jnp
from jax import lax
from jax.experimental import pallas as pl
from jax.experimental.pallas import tpu as pltpu
from jax.experimental.pallas import tpu_sc as plsc

_GSZ = 80
_NB = 5


def _spmm_sc(h, src, dst, w, n, d):
    e = src.shape[0]
    info = plsc.get_sparse_core_info()
    nc, ns = info.num_cores, info.num_subcores
    nw = nc * ns
    ngrp = e // (nw * _GSZ)
    rows_per_tile = 1000
    num_copy_tiles = n // rows_per_tile
    zeros = jnp.zeros((n, d), jnp.float32)
    src2 = src.reshape(e // _GSZ, _GSZ)
    dst2 = dst.reshape(e // _GSZ, _GSZ)
    w2 = w.reshape(e // _GSZ, _GSZ)
    mesh = plsc.VectorSubcoreMesh(core_axis_name="c", subcore_axis_name="s")

    @functools.partial(
        pl.kernel,
        mesh=mesh,
        out_type=jax.ShapeDtypeStruct((nc, n, d), jnp.float32),
        scratch_types=[
            pltpu.VMEM((ngrp, _GSZ), jnp.int32),
            pltpu.VMEM((ngrp, _GSZ), jnp.int32),
            pltpu.VMEM((ngrp, _GSZ), jnp.float32),
            [pltpu.VMEM((_GSZ, d), jnp.float32) for _ in range(_NB)],
            [pltpu.VMEM((_GSZ, d), jnp.float32) for _ in range(_NB)],
            [pltpu.SemaphoreType.DMA for _ in range(_NB)],
            [pltpu.SemaphoreType.DMA for _ in range(_NB)],
            pltpu.VMEM_SHARED((n, d), jnp.float32),
        ],
        compiler_params=pltpu.CompilerParams(use_tc_tiling_on_sc=False),
    )
    def spmm(h_hbm, src_hbm, dst_hbm, w_hbm, z_hbm, out_hbm,
             src_st, dst_st, w_st, rows_bufs, out_bufs, sems, ssems, acc):
        cid = lax.axis_index("c")
        sid = lax.axis_index("s")
        wid = cid * ns + sid
        t0 = wid * ngrp

        pltpu.sync_copy(src_hbm.at[pl.ds(t0, ngrp)], src_st)
        pltpu.sync_copy(dst_hbm.at[pl.ds(t0, ngrp)], dst_st)
        pltpu.sync_copy(w_hbm.at[pl.ds(t0, ngrp)], w_st)
        for b in range(_NB):
            pltpu.async_copy(h_hbm.at[src_st.at[b]], rows_bufs[b], sems[b])

        r0 = pl.multiple_of(sid * rows_per_tile, 8)

        @pl.when(sid < num_copy_tiles)
        def _zero():
            pltpu.sync_copy(z_hbm.at[pl.ds(r0, rows_per_tile)],
                            acc.at[pl.ds(r0, rows_per_tile)])

        plsc.subcore_barrier()

        def outer(p, carry):
            for b in range(_NB):
                j = p * _NB + b
                rows = rows_bufs[b]
                outv = out_bufs[b]
                pltpu.make_async_copy(
                    h_hbm.at[src_st.at[j]], rows, sems[b]).wait()

                @pl.when(j >= _NB)
                def _drain_scatter():
                    pltpu.make_async_copy(
                        outv, acc.at[dst_st.at[j]], ssems[b]).wait()

                for q in range(_GSZ // 16):
                    wv = w_st[j, pl.ds(q * 16, 16)]
                    for ii in range(16):
                        i = q * 16 + ii
                        wsc = wv[ii]
                        for k in range(d // 16):
                            outv[i, pl.ds(k * 16, 16)] = (
                                rows[i, pl.ds(k * 16, 16)] * wsc)
                nxt = j + _NB

                @pl.when(nxt < ngrp)
                def _refill():
                    pltpu.async_copy(
                        h_hbm.at[src_st.at[nxt]], rows, sems[b])

                pltpu.async_copy(
                    outv, acc.at[dst_st.at[j]], ssems[b], add=True)
            return carry

        lax.fori_loop(0, ngrp // _NB, outer, 0)
        for b in range(_NB):
            pltpu.make_async_copy(
                out_bufs[b], acc.at[dst_st.at[0]], ssems[b]).wait()
        plsc.subcore_barrier()

        @pl.when(sid < num_copy_tiles)
        def _copy_out():
            pltpu.sync_copy(acc.at[pl.ds(r0, rows_per_tile)],
                            out_hbm.at[cid, pl.ds(r0, rows_per_tile)])

    return spmm(h, src2, dst2, w2, zeros)


def _mm1_tc(x, w1):
    n, f = x.shape
    h1 = w1.shape[1]
    bm = 1000

    def body(x_ref, w_ref, o_ref):
        o_ref[...] = jnp.dot(x_ref[...], w_ref[...],
                             preferred_element_type=jnp.float32)

    return pl.pallas_call(
        body,
        grid=(n // bm,),
        in_specs=[
            pl.BlockSpec((bm, f), lambda i: (i, 0)),
            pl.BlockSpec((f, h1), lambda i: (0, 0)),
        ],
        out_specs=pl.BlockSpec((bm, h1), lambda i: (i, 0)),
        out_shape=jax.ShapeDtypeStruct((n, h1), jnp.float32),
    )(x, w1)


def _relu_mm2_tc(p0, p1, w2):
    n, h1 = p0.shape
    h2 = w2.shape[1]

    def body(p0_ref, p1_ref, w_ref, o_ref):
        h = jnp.maximum(p0_ref[...] + p1_ref[...], 0.0)
        o_ref[...] = jnp.dot(h, w_ref[...],
                             preferred_element_type=jnp.float32)

    return pl.pallas_call(
        body,
        out_shape=jax.ShapeDtypeStruct((n, h2), jnp.float32),
    )(p0, p1, w2)


def _decoder_tc(q0, q1):
    n, h2 = q0.shape
    bm = 128
    grid = (n + bm - 1) // bm

    def body(q0b_ref, q1b_ref, q0f_ref, q1f_ref, o_ref, m_ref, zf_ref):
        i = pl.program_id(0)

        @pl.when(i == 0)
        def _build_zf():
            zf_ref[...] = q0f_ref[...] + q1f_ref[...]

        zb = q0b_ref[...] + q1b_ref[...]
        m_ref[...] = lax.dot_general(
            zb, zf_ref[...], (((1,), (1,)), ((), ())),
            preferred_element_type=jnp.float32)
        for r in range(bm):
            o_ref[pl.ds(r * n, n)] = m_ref[r, :]

    out = pl.pallas_call(
        body,
        grid=(grid,),
        in_specs=[
            pl.BlockSpec((bm, h2), lambda i: (i, 0)),
            pl.BlockSpec((bm, h2), lambda i: (i, 0)),
            pl.BlockSpec((n, h2), lambda i: (0, 0)),
            pl.BlockSpec((n, h2), lambda i: (0, 0)),
        ],
        out_specs=pl.BlockSpec((bm * n,), lambda i: (i,)),
        out_shape=jax.ShapeDtypeStruct((n * n,), jnp.float32),
        scratch_shapes=[pltpu.VMEM((bm, n), jnp.float32),
                        pltpu.VMEM((n, h2), jnp.float32)],
    )(q0, q1, q0, q1)
    return out


def kernel(x, edge_index, edge_weight, W1, W2):
    n = x.shape[0]
    ei = edge_index.astype(jnp.int32)
    src = ei[0]
    dst = ei[1]

    mm1 = _mm1_tc(x, W1)
    p = _spmm_sc(mm1, src, dst, edge_weight, n, W1.shape[1])
    y = _relu_mm2_tc(p[0], p[1], W2)
    q = _spmm_sc(y, src, dst, edge_weight, n, W2.shape[1])
    return _decoder_tc(q[0], q[1])


# speedup vs baseline: 1.2788x; 1.0435x over previous
import functools

import jax
import jax.numpy as jnp
from jax import lax
from jax.experimental import pallas as pl
from jax.experimental.pallas import tpu as pltpu
from jax.experimental.pallas import tpu_sc as plsc

_GSZ = 80
_NB = 5


def _spmm_sc(h, src, dst, w, n, d):
    e = src.shape[0]
    info = plsc.get_sparse_core_info()
    nc, ns = info.num_cores, info.num_subcores
    nw = nc * ns
    ngrp = e // (nw * _GSZ)
    rows_per_tile = 1000
    num_copy_tiles = n // rows_per_tile
    zeros = jnp.zeros((n, d), jnp.float32)
    src2 = src.reshape(e // _GSZ, _GSZ)
    dst2 = dst.reshape(e // _GSZ, _GSZ)
    w2 = w.reshape(e // _GSZ, _GSZ)
    mesh = plsc.VectorSubcoreMesh(core_axis_name="c", subcore_axis_name="s")

    @functools.partial(
        pl.kernel,
        mesh=mesh,
        out_type=jax.ShapeDtypeStruct((nc, n, d), jnp.float32),
        scratch_types=[
            pltpu.VMEM((ngrp, _GSZ), jnp.int32),
            pltpu.VMEM((ngrp, _GSZ), jnp.int32),
            pltpu.VMEM((ngrp, _GSZ), jnp.float32),
            [pltpu.VMEM((_GSZ, d), jnp.float32) for _ in range(_NB)],
            [pltpu.VMEM((_GSZ, d), jnp.float32) for _ in range(_NB)],
            [pltpu.SemaphoreType.DMA for _ in range(_NB)],
            [pltpu.SemaphoreType.DMA for _ in range(_NB)],
            pltpu.VMEM_SHARED((n, d), jnp.float32),
        ],
        compiler_params=pltpu.CompilerParams(use_tc_tiling_on_sc=False),
    )
    def spmm(h_hbm, src_hbm, dst_hbm, w_hbm, z_hbm, out_hbm,
             src_st, dst_st, w_st, rows_bufs, out_bufs, sems, ssems, acc):
        cid = lax.axis_index("c")
        sid = lax.axis_index("s")
        wid = cid * ns + sid
        t0 = wid * ngrp

        pltpu.sync_copy(src_hbm.at[pl.ds(t0, ngrp)], src_st)
        pltpu.sync_copy(dst_hbm.at[pl.ds(t0, ngrp)], dst_st)
        pltpu.sync_copy(w_hbm.at[pl.ds(t0, ngrp)], w_st)
        for b in range(_NB):
            pltpu.async_copy(h_hbm.at[src_st.at[b]], rows_bufs[b], sems[b])

        r0 = pl.multiple_of(sid * rows_per_tile, 8)

        @pl.when(sid < num_copy_tiles)
        def _zero():
            pltpu.sync_copy(z_hbm.at[pl.ds(r0, rows_per_tile)],
                            acc.at[pl.ds(r0, rows_per_tile)])

        plsc.subcore_barrier()

        def outer(p, carry):
            for b in range(_NB):
                j = p * _NB + b
                rows = rows_bufs[b]
                outv = out_bufs[b]
                pltpu.make_async_copy(
                    h_hbm.at[src_st.at[j]], rows, sems[b]).wait()

                @pl.when(j >= _NB)
                def _drain_scatter():
                    pltpu.make_async_copy(
                        outv, acc.at[dst_st.at[j]], ssems[b]).wait()

                for q in range(_GSZ // 16):
                    wv = w_st[j, pl.ds(q * 16, 16)]
                    for ii in range(16):
                        i = q * 16 + ii
                        wsc = wv[ii]
                        for k in range(d // 16):
                            outv[i, pl.ds(k * 16, 16)] = (
                                rows[i, pl.ds(k * 16, 16)] * wsc)
                nxt = j + _NB

                @pl.when(nxt < ngrp)
                def _refill():
                    pltpu.async_copy(
                        h_hbm.at[src_st.at[nxt]], rows, sems[b])

                pltpu.async_copy(
                    outv, acc.at[dst_st.at[j]], ssems[b], add=True)
            return carry

        lax.fori_loop(0, ngrp // _NB, outer, 0)
        for b in range(_NB):
            pltpu.make_async_copy(
                out_bufs[b], acc.at[dst_st.at[0]], ssems[b]).wait()
        plsc.subcore_barrier()

        @pl.when(sid < num_copy_tiles)
        def _copy_out():
            pltpu.sync_copy(acc.at[pl.ds(r0, rows_per_tile)],
                            out_hbm.at[cid, pl.ds(r0, rows_per_tile)])

    return spmm(h, src2, dst2, w2, zeros)


def _mm1_tc(x, w1):
    n, f = x.shape
    h1 = w1.shape[1]
    bm = 1000

    def body(x_ref, w_ref, o_ref):
        o_ref[...] = jnp.dot(x_ref[...], w_ref[...],
                             preferred_element_type=jnp.float32)

    return pl.pallas_call(
        body,
        grid=(n // bm,),
        in_specs=[
            pl.BlockSpec((bm, f), lambda i: (i, 0)),
            pl.BlockSpec((f, h1), lambda i: (0, 0)),
        ],
        out_specs=pl.BlockSpec((bm, h1), lambda i: (i, 0)),
        out_shape=jax.ShapeDtypeStruct((n, h1), jnp.float32),
    )(x, w1)


def _relu_mm2_tc(p0, p1, w2):
    n, h1 = p0.shape
    h2 = w2.shape[1]

    def body(p0_ref, p1_ref, w_ref, o_ref):
        h = jnp.maximum(p0_ref[...] + p1_ref[...], 0.0)
        o_ref[...] = jnp.dot(h, w_ref[...],
                             preferred_element_type=jnp.float32)

    return pl.pallas_call(
        body,
        out_shape=jax.ShapeDtypeStruct((n, h2), jnp.float32),
    )(p0, p1, w2)


def _decoder_tc(q0, q1):
    n, h2 = q0.shape
    bm = 256
    grid = (n + bm - 1) // bm

    def body(q0b_ref, q1b_ref, q0f_ref, q1f_ref, o_ref, m_ref, zf_ref):
        i = pl.program_id(0)

        @pl.when(i == 0)
        def _build_zf():
            zf_ref[...] = q0f_ref[...] + q1f_ref[...]

        zb = q0b_ref[...] + q1b_ref[...]
        m_ref[...] = lax.dot_general(
            zb, zf_ref[...], (((1,), (1,)), ((), ())),
            preferred_element_type=jnp.float32)
        for r in range(bm):
            o_ref[pl.ds(r * n, n)] = m_ref[r, :]

    out = pl.pallas_call(
        body,
        grid=(grid,),
        in_specs=[
            pl.BlockSpec((bm, h2), lambda i: (i, 0)),
            pl.BlockSpec((bm, h2), lambda i: (i, 0)),
            pl.BlockSpec((n, h2), lambda i: (0, 0)),
            pl.BlockSpec((n, h2), lambda i: (0, 0)),
        ],
        out_specs=pl.BlockSpec((bm * n,), lambda i: (i,)),
        out_shape=jax.ShapeDtypeStruct((n * n,), jnp.float32),
        scratch_shapes=[pltpu.VMEM((bm, n), jnp.float32),
                        pltpu.VMEM((n, h2), jnp.float32)],
    )(q0, q1, q0, q1)
    return out


def kernel(x, edge_index, edge_weight, W1, W2):
    n = x.shape[0]
    ei = edge_index.astype(jnp.int32)
    src = ei[0]
    dst = ei[1]

    mm1 = _mm1_tc(x, W1)
    p = _spmm_sc(mm1, src, dst, edge_weight, n, W1.shape[1])
    y = _relu_mm2_tc(p[0], p[1], W2)
    q = _spmm_sc(y, src, dst, edge_weight, n, W2.shape[1])
    return _decoder_tc(q[0], q[1])


# speedup vs baseline: 1.3394x; 1.0474x over previous
import functools

import jax
import jax.numpy as jnp
from jax import lax
from jax.experimental import pallas as pl
from jax.experimental.pallas import tpu as pltpu
from jax.experimental.pallas import tpu_sc as plsc

_GSZ = 80
_NB = 5


def _spmm_sc(h, src, dst, w, n, d):
    e = src.shape[0]
    info = plsc.get_sparse_core_info()
    nc, ns = info.num_cores, info.num_subcores
    nw = nc * ns
    ngrp = e // (nw * _GSZ)
    rows_per_tile = 1000
    num_copy_tiles = n // rows_per_tile
    zeros = jnp.zeros((n, d), jnp.float32)
    src2 = src.reshape(e // _GSZ, _GSZ)
    dst2 = dst.reshape(e // _GSZ, _GSZ)
    w2 = w.reshape(e // _GSZ, _GSZ)
    mesh = plsc.VectorSubcoreMesh(core_axis_name="c", subcore_axis_name="s")

    @functools.partial(
        pl.kernel,
        mesh=mesh,
        out_type=jax.ShapeDtypeStruct((nc, n, d), jnp.float32),
        scratch_types=[
            pltpu.VMEM((ngrp, _GSZ), jnp.int32),
            pltpu.VMEM((ngrp, _GSZ), jnp.int32),
            pltpu.VMEM((ngrp, _GSZ), jnp.float32),
            [pltpu.VMEM((_GSZ, d), jnp.float32) for _ in range(_NB)],
            [pltpu.VMEM((_GSZ, d), jnp.float32) for _ in range(_NB)],
            [pltpu.SemaphoreType.DMA for _ in range(_NB)],
            [pltpu.SemaphoreType.DMA for _ in range(_NB)],
            pltpu.VMEM_SHARED((n, d), jnp.float32),
        ],
        compiler_params=pltpu.CompilerParams(use_tc_tiling_on_sc=False),
    )
    def spmm(h_hbm, src_hbm, dst_hbm, w_hbm, z_hbm, out_hbm,
             src_st, dst_st, w_st, rows_bufs, out_bufs, sems, ssems, acc):
        cid = lax.axis_index("c")
        sid = lax.axis_index("s")
        wid = cid * ns + sid
        t0 = wid * ngrp

        pltpu.sync_copy(src_hbm.at[pl.ds(t0, ngrp)], src_st)
        pltpu.sync_copy(dst_hbm.at[pl.ds(t0, ngrp)], dst_st)
        pltpu.sync_copy(w_hbm.at[pl.ds(t0, ngrp)], w_st)
        for b in range(_NB):
            pltpu.async_copy(h_hbm.at[src_st.at[b]], rows_bufs[b], sems[b])

        r0 = pl.multiple_of(sid * rows_per_tile, 8)

        @pl.when(sid < num_copy_tiles)
        def _zero():
            pltpu.sync_copy(z_hbm.at[pl.ds(r0, rows_per_tile)],
                            acc.at[pl.ds(r0, rows_per_tile)])

        plsc.subcore_barrier()

        def outer(p, carry):
            for b in range(_NB):
                j = p * _NB + b
                rows = rows_bufs[b]
                outv = out_bufs[b]
                pltpu.make_async_copy(
                    h_hbm.at[src_st.at[j]], rows, sems[b]).wait()

                @pl.when(j >= _NB)
                def _drain_scatter():
                    pltpu.make_async_copy(
                        outv, acc.at[dst_st.at[j]], ssems[b]).wait()

                for q in range(_GSZ // 16):
                    wv = w_st[j, pl.ds(q * 16, 16)]
                    for ii in range(16):
                        i = q * 16 + ii
                        wsc = wv[ii]
                        for k in range(d // 16):
                            outv[i, pl.ds(k * 16, 16)] = (
                                rows[i, pl.ds(k * 16, 16)] * wsc)
                nxt = j + _NB

                @pl.when(nxt < ngrp)
                def _refill():
                    pltpu.async_copy(
                        h_hbm.at[src_st.at[nxt]], rows, sems[b])

                pltpu.async_copy(
                    outv, acc.at[dst_st.at[j]], ssems[b], add=True)
            return carry

        lax.fori_loop(0, ngrp // _NB, outer, 0)
        for b in range(_NB):
            pltpu.make_async_copy(
                out_bufs[b], acc.at[dst_st.at[0]], ssems[b]).wait()
        plsc.subcore_barrier()

        @pl.when(sid < num_copy_tiles)
        def _copy_out():
            pltpu.sync_copy(acc.at[pl.ds(r0, rows_per_tile)],
                            out_hbm.at[cid, pl.ds(r0, rows_per_tile)])

    return spmm(h, src2, dst2, w2, zeros)


def _mm1_tc(x, w1):
    n, f = x.shape
    h1 = w1.shape[1]
    bm = 1000

    def body(x_ref, w_ref, o_ref):
        o_ref[...] = jnp.dot(x_ref[...], w_ref[...],
                             preferred_element_type=jnp.float32)

    return pl.pallas_call(
        body,
        grid=(n // bm,),
        in_specs=[
            pl.BlockSpec((bm, f), lambda i: (i, 0)),
            pl.BlockSpec((f, h1), lambda i: (0, 0)),
        ],
        out_specs=pl.BlockSpec((bm, h1), lambda i: (i, 0)),
        out_shape=jax.ShapeDtypeStruct((n, h1), jnp.float32),
    )(x, w1)


def _relu_mm2_tc(p, w2):
    _, n, h1 = p.shape
    h2 = w2.shape[1]

    def body(p_ref, w_ref, o_ref):
        h = jnp.maximum(p_ref[0] + p_ref[1], 0.0)
        o_ref[...] = jnp.dot(h, w_ref[...],
                             preferred_element_type=jnp.float32)

    return pl.pallas_call(
        body,
        out_shape=jax.ShapeDtypeStruct((n, h2), jnp.float32),
    )(p, w2)


def _decoder_tc(q):
    _, n, h2 = q.shape
    bm = 256
    grid = (n + bm - 1) // bm

    def body(qb_ref, qf_ref, o_ref, m_ref, zf_ref):
        i = pl.program_id(0)

        @pl.when(i == 0)
        def _build_zf():
            zf_ref[...] = qf_ref[0] + qf_ref[1]

        zb = qb_ref[0] + qb_ref[1]
        m_ref[...] = lax.dot_general(
            zb, zf_ref[...], (((1,), (1,)), ((), ())),
            preferred_element_type=jnp.float32)
        for r in range(bm):
            o_ref[pl.ds(r * n, n)] = m_ref[r, :]

    out = pl.pallas_call(
        body,
        grid=(grid,),
        in_specs=[
            pl.BlockSpec((2, bm, h2), lambda i: (0, i, 0)),
            pl.BlockSpec((2, n, h2), lambda i: (0, 0, 0)),
        ],
        out_specs=pl.BlockSpec((bm * n,), lambda i: (i,)),
        out_shape=jax.ShapeDtypeStruct((n * n,), jnp.float32),
        scratch_shapes=[pltpu.VMEM((bm, n), jnp.float32),
                        pltpu.VMEM((n, h2), jnp.float32)],
    )(q, q)
    return out


def kernel(x, edge_index, edge_weight, W1, W2):
    n = x.shape[0]
    ei = edge_index.astype(jnp.int32)
    src = ei[0]
    dst = ei[1]

    mm1 = _mm1_tc(x, W1)
    p = _spmm_sc(mm1, src, dst, edge_weight, n, W1.shape[1])
    y = _relu_mm2_tc(p, W2)
    q = _spmm_sc(y, src, dst, edge_weight, n, W2.shape[1])
    return _decoder_tc(q)


# speedup vs baseline: 1.3446x; 1.0039x over previous
import functools

import jax
import jax.numpy as jnp
from jax import lax
from jax.experimental import pallas as pl
from jax.experimental.pallas import tpu as pltpu
from jax.experimental.pallas import tpu_sc as plsc

_GSZ = 80
_NB = 5


def _spmm_sc(h, src, dst, w, n, d):
    e = src.shape[0]
    info = plsc.get_sparse_core_info()
    nc, ns = info.num_cores, info.num_subcores
    nw = nc * ns
    ngrp = e // (nw * _GSZ)
    rows_per_tile = 1000
    num_copy_tiles = n // rows_per_tile
    src2 = src.reshape(e // _GSZ, _GSZ)
    dst2 = dst.reshape(e // _GSZ, _GSZ)
    w2 = w.reshape(e // _GSZ, _GSZ)
    mesh = plsc.VectorSubcoreMesh(core_axis_name="c", subcore_axis_name="s")

    @functools.partial(
        pl.kernel,
        mesh=mesh,
        out_type=jax.ShapeDtypeStruct((nc, n, d), jnp.float32),
        scratch_types=[
            pltpu.VMEM((ngrp, _GSZ), jnp.int32),
            pltpu.VMEM((ngrp, _GSZ), jnp.int32),
            pltpu.VMEM((ngrp, _GSZ), jnp.float32),
            [pltpu.VMEM((_GSZ, d), jnp.float32) for _ in range(_NB)],
            [pltpu.VMEM((_GSZ, d), jnp.float32) for _ in range(_NB)],
            [pltpu.SemaphoreType.DMA for _ in range(_NB)],
            [pltpu.SemaphoreType.DMA for _ in range(_NB)],
            pltpu.VMEM_SHARED((n, d), jnp.float32),
        ],
        compiler_params=pltpu.CompilerParams(use_tc_tiling_on_sc=False),
    )
    def spmm(h_hbm, src_hbm, dst_hbm, w_hbm, out_hbm,
             src_st, dst_st, w_st, rows_bufs, out_bufs, sems, ssems, acc):
        cid = lax.axis_index("c")
        sid = lax.axis_index("s")
        wid = cid * ns + sid
        t0 = wid * ngrp

        pltpu.sync_copy(src_hbm.at[pl.ds(t0, ngrp)], src_st)
        pltpu.sync_copy(dst_hbm.at[pl.ds(t0, ngrp)], dst_st)
        pltpu.sync_copy(w_hbm.at[pl.ds(t0, ngrp)], w_st)
        for b in range(_NB):
            pltpu.async_copy(h_hbm.at[src_st.at[b]], rows_bufs[b], sems[b])

        r0 = pl.multiple_of(sid * rows_per_tile, 8)
        zbuf = out_bufs[0]
        for rr in range(_GSZ):
            for k in range(d // 16):
                zbuf[rr, pl.ds(k * 16, 16)] = jnp.zeros((16,), jnp.float32)

        @pl.when(sid < num_copy_tiles)
        def _zero():
            for cpy in range(rows_per_tile // _GSZ):
                pltpu.sync_copy(zbuf,
                                acc.at[pl.ds(r0 + cpy * _GSZ, _GSZ)])
            rem = rows_per_tile % _GSZ
            if rem:
                pltpu.sync_copy(
                    zbuf.at[pl.ds(0, rem)],
                    acc.at[pl.ds(r0 + rows_per_tile - rem, rem)])

        plsc.subcore_barrier()

        def outer(p, carry):
            for b in range(_NB):
                j = p * _NB + b
                rows = rows_bufs[b]
                outv = out_bufs[b]
                pltpu.make_async_copy(
                    h_hbm.at[src_st.at[j]], rows, sems[b]).wait()

                @pl.when(j >= _NB)
                def _drain_scatter():
                    pltpu.make_async_copy(
                        outv, acc.at[dst_st.at[j]], ssems[b]).wait()

                for q in range(_GSZ // 16):
                    wv = w_st[j, pl.ds(q * 16, 16)]
                    for ii in range(16):
                        i = q * 16 + ii
                        wsc = wv[ii]
                        for k in range(d // 16):
                            outv[i, pl.ds(k * 16, 16)] = (
                                rows[i, pl.ds(k * 16, 16)] * wsc)
                nxt = j + _NB

                @pl.when(nxt < ngrp)
                def _refill():
                    pltpu.async_copy(
                        h_hbm.at[src_st.at[nxt]], rows, sems[b])

                pltpu.async_copy(
                    outv, acc.at[dst_st.at[j]], ssems[b], add=True)
            return carry

        lax.fori_loop(0, ngrp // _NB, outer, 0)
        for b in range(_NB):
            pltpu.make_async_copy(
                out_bufs[b], acc.at[dst_st.at[0]], ssems[b]).wait()
        plsc.subcore_barrier()

        @pl.when(sid < num_copy_tiles)
        def _copy_out():
            pltpu.sync_copy(acc.at[pl.ds(r0, rows_per_tile)],
                            out_hbm.at[cid, pl.ds(r0, rows_per_tile)])

    return spmm(h, src2, dst2, w2)


def _mm1_tc(x, w1):
    n, f = x.shape
    h1 = w1.shape[1]
    bm = 1000

    def body(x_ref, w_ref, o_ref):
        o_ref[...] = jnp.dot(x_ref[...], w_ref[...],
                             preferred_element_type=jnp.float32)

    return pl.pallas_call(
        body,
        grid=(n // bm,),
        in_specs=[
            pl.BlockSpec((bm, f), lambda i: (i, 0)),
            pl.BlockSpec((f, h1), lambda i: (0, 0)),
        ],
        out_specs=pl.BlockSpec((bm, h1), lambda i: (i, 0)),
        out_shape=jax.ShapeDtypeStruct((n, h1), jnp.float32),
    )(x, w1)


def _relu_mm2_tc(p, w2):
    _, n, h1 = p.shape
    h2 = w2.shape[1]

    def body(p_ref, w_ref, o_ref):
        h = jnp.maximum(p_ref[0] + p_ref[1], 0.0)
        o_ref[...] = jnp.dot(h, w_ref[...],
                             preferred_element_type=jnp.float32)

    return pl.pallas_call(
        body,
        out_shape=jax.ShapeDtypeStruct((n, h2), jnp.float32),
    )(p, w2)


def _decoder_tc(q):
    _, n, h2 = q.shape
    bm = 256
    grid = (n + bm - 1) // bm

    def body(qb_ref, qf_ref, o_ref, m_ref, zf_ref):
        i = pl.program_id(0)

        @pl.when(i == 0)
        def _build_zf():
            zf_ref[...] = qf_ref[0] + qf_ref[1]

        zb = qb_ref[0] + qb_ref[1]
        m_ref[...] = lax.dot_general(
            zb, zf_ref[...], (((1,), (1,)), ((), ())),
            preferred_element_type=jnp.float32)
        for r in range(bm):
            o_ref[pl.ds(r * n, n)] = m_ref[r, :]

    out = pl.pallas_call(
        body,
        grid=(grid,),
        in_specs=[
            pl.BlockSpec((2, bm, h2), lambda i: (0, i, 0)),
            pl.BlockSpec((2, n, h2), lambda i: (0, 0, 0)),
        ],
        out_specs=pl.BlockSpec((bm * n,), lambda i: (i,)),
        out_shape=jax.ShapeDtypeStruct((n * n,), jnp.float32),
        scratch_shapes=[pltpu.VMEM((bm, n), jnp.float32),
                        pltpu.VMEM((n, h2), jnp.float32)],
    )(q, q)
    return out


def kernel(x, edge_index, edge_weight, W1, W2):
    n = x.shape[0]
    ei = edge_index.astype(jnp.int32)
    src = ei[0]
    dst = ei[1]

    mm1 = _mm1_tc(x, W1)
    p = _spmm_sc(mm1, src, dst, edge_weight, n, W1.shape[1])
    y = _relu_mm2_tc(p, W2)
    q = _spmm_sc(y, src, dst, edge_weight, n, W2.shape[1])
    return _decoder_tc(q)
